# baseline (speedup 1.0000x reference)
import functools

import jax
import jax.numpy as jnp
from jax import lax
from jax.experimental import pallas as pl
from jax.experimental.pallas import tpu as pltpu
from jax.experimental.pallas import tpu_sc as plsc

N = 10000
E = 320000
EMB = 128
NEF = 3

NC = 2
NS = 16
NW = NC * NS

SINK = 112
NP = N + SINK
RPS = NP // NS

EP = 327680
EROWS = EP // 128
RPT = EROWS // NS

RB = 1000
EB = 2048


def _sc_agg(t_hbm, m_hbm, src_hbm, dst_hbm, parts_hbm,
            acc_sh, srcb, dstb, gxb, gsem, ssem):
    c = lax.axis_index("c")
    s = lax.axis_index("s")

    def zrow(i, carry):
        for jj in range(8):
            gxb[i, pl.ds(jj * 16, 16)] = jnp.zeros((16,), jnp.float32)
        return carry
    lax.fori_loop(0, 128, zrow, 0)
    nfull = RPS // 128
    def zcopy(i, carry):
        pltpu.sync_copy(gxb, acc_sh.at[pl.ds(s * RPS + i * 128, 128)])
        return carry
    lax.fori_loop(0, nfull, zcopy, 0)
    pltpu.sync_copy(gxb.at[pl.ds(0, RPS - nfull * 128)],
                    acc_sh.at[pl.ds(s * RPS + nfull * 128, RPS - nfull * 128)])
    plsc.subcore_barrier()

    def block(i, carry):
        rb = s * RPT + i
        pltpu.sync_copy(dst_hbm.at[pl.ds(rb, 1)], dstb)

        @pl.when(c == 0)
        def _():
            pltpu.sync_copy(src_hbm.at[pl.ds(rb, 1)], srcb)
            pltpu.async_copy(t_hbm.at[srcb.at[0]], gxb, gsem).wait()

        @pl.when(c == 1)
        def _():
            pltpu.sync_copy(m_hbm.at[pl.ds(rb * 128, 128)], gxb)

        pltpu.async_copy(gxb, acc_sh.at[dstb.at[0]], ssem, add=True).wait()
        return carry
    lax.fori_loop(0, RPT, block, 0)
    plsc.subcore_barrier()

    def ocopy(i, carry):
        pltpu.sync_copy(acc_sh.at[pl.ds(s * RPS + i * 128, 128)],
                        parts_hbm.at[c, pl.ds(s * RPS + i * 128, 128)])
        return carry
    lax.fori_loop(0, nfull, ocopy, 0)
    pltpu.sync_copy(acc_sh.at[pl.ds(s * RPS + nfull * 128, RPS - nfull * 128)],
                    parts_hbm.at[c, pl.ds(s * RPS + nfull * 128, RPS - nfull * 128)])


def _bdot(a, b):
    return jnp.dot(a.astype(jnp.bfloat16), b.astype(jnp.bfloat16),
                   preferred_element_type=jnp.float32)


def _tc_tab(xp_ref, Winp_ref, bin_ref, t_ref):
    t_ref[...] = _bdot(xp_ref[...], Winp_ref[...]) + bin_ref[...]


def _tc_emb(eavp_ref, Wep_ref, bep_ref, m_ref):
    m_ref[...] = _bdot(eavp_ref[...], Wep_ref[...]) + bep_ref[...]


def _tc_z(parts_ref, t_ref, Wep_ref, bep_ref, W1_ref, b1_ref, z_ref, stats_ref):
    i = pl.program_id(0)
    col = lax.broadcasted_iota(jnp.int32, (1, 16), 1)
    srow = jnp.where(col == 1, 1.0, 0.0).astype(jnp.float32)
    embself = _bdot(srow, Wep_ref[...]) + bep_ref[...]
    aggr = jnp.concatenate(
        [parts_ref[0] + t_ref[...], parts_ref[1] + embself], axis=1)
    z = _bdot(aggr, W1_ref[...]) + b1_ref[...]
    z_ref[...] = z

    @pl.when(i == 0)
    def _():
        stats_ref[...] = jnp.zeros_like(stats_ref)
    stats_ref[0:1, :] += jnp.sum(z, axis=0, keepdims=True)
    stats_ref[1:2, :] += jnp.sum(z * z, axis=0, keepdims=True)


def _tc_norm(relu_out, z_ref, stats_ref, g_ref, bt_ref, W2_ref, b2_ref, o_ref):
    mu = stats_ref[0:1, :] / N
    var = stats_ref[1:2, :] / N - mu * mu
    rstd = lax.rsqrt(var + 1e-5)
    zn = (z_ref[...] - mu) * rstd * g_ref[...] + bt_ref[...]
    zn = jnp.maximum(zn, 0.0)
    o = _bdot(zn, W2_ref[...]) + b2_ref[...]
    if relu_out:
        o = jnp.maximum(o, 0.0)
    o_ref[...] = o


def _full(shape):
    return pl.BlockSpec(shape, lambda i: (0,) * len(shape))


@functools.cache
def _sc_kernels():
    mesh = plsc.VectorSubcoreMesh(core_axis_name="c", subcore_axis_name="s")
    agg = pl.kernel(
        _sc_agg,
        out_type=jax.ShapeDtypeStruct((NC, NP, EMB), jnp.float32),
        mesh=mesh,
        compiler_params=pltpu.CompilerParams(use_tc_tiling_on_sc=False),
        scratch_types=[
            pltpu.VMEM_SHARED((NP, EMB), jnp.float32),
            pltpu.VMEM((1, 128), jnp.int32),
            pltpu.VMEM((1, 128), jnp.int32),
            pltpu.VMEM((128, EMB), jnp.float32),
            pltpu.SemaphoreType.DMA,
            pltpu.SemaphoreType.DMA,
        ],
    )
    return agg


_GRID = N // RB


def _node_table(xp, Winp, bin2):
    return pl.pallas_call(
        _tc_tab,
        grid=(_GRID,),
        in_specs=[pl.BlockSpec((RB, 16), lambda i: (i, 0)),
                  _full((16, EMB)), _full((1, EMB))],
        out_specs=pl.BlockSpec((RB, EMB), lambda i: (i, 0)),
        out_shape=jax.ShapeDtypeStruct((N, EMB), jnp.float32),
    )(xp, Winp, bin2)


def _edge_table(eavp, Wep, bep):
    return pl.pallas_call(
        _tc_emb,
        grid=(EP // EB,),
        in_specs=[pl.BlockSpec((EB, 16), lambda i: (i, 0)),
                  _full((16, EMB)), _full((1, EMB))],
        out_specs=pl.BlockSpec((EB, EMB), lambda i: (i, 0)),
        out_shape=jax.ShapeDtypeStruct((EP, EMB), jnp.float32),
    )(eavp, Wep, bep)


def _dense(parts, t, Wep, bep, W1, b1, g, bt, W2, b2, relu_out):
    z, stats = pl.pallas_call(
        _tc_z,
        grid=(_GRID,),
        in_specs=[
            pl.BlockSpec((NC, RB, EMB), lambda i: (0, i, 0)),
            pl.BlockSpec((RB, EMB), lambda i: (i, 0)),
            _full((16, EMB)), _full((1, EMB)),
            _full((2 * EMB, 2 * EMB)), _full((1, 2 * EMB)),
        ],
        out_specs=[
            pl.BlockSpec((RB, 2 * EMB), lambda i: (i, 0)),
            _full((2, 2 * EMB)),
        ],
        out_shape=[
            jax.ShapeDtypeStruct((N, 2 * EMB), jnp.float32),
            jax.ShapeDtypeStruct((2, 2 * EMB), jnp.float32),
        ],
    )(parts, t, Wep, bep, W1, b1)
    return pl.pallas_call(
        functools.partial(_tc_norm, relu_out),
        grid=(_GRID,),
        in_specs=[
            pl.BlockSpec((RB, 2 * EMB), lambda i: (i, 0)),
            _full((2, 2 * EMB)),
            _full((1, 2 * EMB)), _full((1, 2 * EMB)),
            _full((2 * EMB, EMB)), _full((1, EMB)),
        ],
        out_specs=pl.BlockSpec((RB, EMB), lambda i: (i, 0)),
        out_shape=jax.ShapeDtypeStruct((N, EMB), jnp.float32),
    )(z, stats, g, bt, W2, b2)


def _pad16(w):
    return jnp.concatenate([w, jnp.zeros((16 - NEF, EMB), jnp.float32)], axis=0)


def kernel(x, edge_index, edge_attr, Win, bin_, We0, be0, W10, b10, g0, bt0,
           W20, b20, We1, be1, W11, b11, g1, bt1, W21, b21):
    f32 = jnp.float32
    pad = EP - E
    lane = jnp.arange(pad, dtype=jnp.int32) % 16
    src = jnp.concatenate([edge_index[0], lane]).reshape(EROWS, 128)
    dst = jnp.concatenate([edge_index[1], N + lane]).reshape(EROWS, 128)
    xp = jnp.concatenate([x, jnp.zeros((N, 16 - NEF), f32)], axis=1)
    eavp = jnp.concatenate([
        jnp.concatenate([edge_attr, jnp.zeros((E, 16 - NEF), f32)], axis=1),
        jnp.zeros((pad, 16), f32)], axis=0)

    agg = _sc_kernels()
    t0 = _node_table(xp, _pad16(Win), bin_.reshape(1, EMB))
    m0 = _edge_table(eavp, _pad16(We0), be0.reshape(1, EMB))
    parts0 = agg(t0, m0, src, dst)[:, :N, :]
    h1 = _dense(parts0, t0, _pad16(We0), be0.reshape(1, EMB), W10,
                b10.reshape(1, 2 * EMB), g0.reshape(1, 2 * EMB),
                bt0.reshape(1, 2 * EMB), W20, b20.reshape(1, EMB), True)
    m1 = _edge_table(eavp, _pad16(We1), be1.reshape(1, EMB))
    parts1 = agg(h1, m1, src, dst)[:, :N, :]
    out = _dense(parts1, h1, _pad16(We1), be1.reshape(1, EMB), W11,
                 b11.reshape(1, 2 * EMB), g1.reshape(1, 2 * EMB),
                 bt1.reshape(1, 2 * EMB), W21, b21.reshape(1, EMB), False)
    return out


# speedup vs baseline: 1.3581x; 1.3581x over previous
import functools

import jax
import jax.numpy as jnp
from jax import lax
from jax.experimental import pallas as pl
from jax.experimental.pallas import tpu as pltpu
from jax.experimental.pallas import tpu_sc as plsc

N = 10000
E = 320000
EMB = 128
NEF = 3

NC = 2
NS = 16
NW = NC * NS

SINK = 112
NP = N + SINK
RPS = NP // NS

EP = 327680
EROWS = EP // 128
RPT = EROWS // NS

RB = 1000
EB = 2048


def _sc_agg(t_hbm, m_hbm, src_hbm, dst_hbm, parts_hbm,
            acc_sh, srcb, dstb, gxb, isem, gsem, ssem):
    c = lax.axis_index("c")
    s = lax.axis_index("s")

    def zrow(i, carry):
        for jj in range(8):
            gxb[i, pl.ds(jj * 16, 16)] = jnp.zeros((16,), jnp.float32)
        return carry
    lax.fori_loop(0, 128, zrow, 0)
    nfull = RPS // 128
    def zcopy(i, carry):
        pltpu.sync_copy(gxb.at[pl.ds(0, 128)],
                        acc_sh.at[pl.ds(s * RPS + i * 128, 128)])
        return carry
    lax.fori_loop(0, nfull, zcopy, 0)
    pltpu.sync_copy(gxb.at[pl.ds(0, RPS - nfull * 128)],
                    acc_sh.at[pl.ds(s * RPS + nfull * 128, RPS - nfull * 128)])
    plsc.subcore_barrier()

    base = s * RPT
    nt = RPT // 2
    srcs = (srcb.at[pl.ds(0, 1)], srcb.at[pl.ds(1, 1)])
    dsts = (dstb.at[pl.ds(0, 1)], dstb.at[pl.ds(1, 1)])
    gxs = (gxb.at[pl.ds(0, 128)], gxb.at[pl.ds(128, 128)])

    @pl.when(c == 0)
    def _():
        pltpu.async_copy(src_hbm.at[pl.ds(base, 1)], srcs[0], isem)
        pltpu.async_copy(dst_hbm.at[pl.ds(base, 1)], dsts[0], isem)

        def block(t, carry):
            r0 = base + 2 * t
            pltpu.make_async_copy(src_hbm.at[pl.ds(r0, 1)], srcs[0], isem).wait()
            pltpu.make_async_copy(dst_hbm.at[pl.ds(r0, 1)], dsts[0], isem).wait()
            i1s = pltpu.async_copy(src_hbm.at[pl.ds(r0 + 1, 1)], srcs[1], isem)
            i1d = pltpu.async_copy(dst_hbm.at[pl.ds(r0 + 1, 1)], dsts[1], isem)
            g0 = pltpu.async_copy(t_hbm.at[srcs[0].at[0]], gxs[0], gsem)
            g0.wait()
            s0 = pltpu.async_copy(gxs[0], acc_sh.at[dsts[0].at[0]], ssem, add=True)
            i1s.wait()
            i1d.wait()
            g1 = pltpu.async_copy(t_hbm.at[srcs[1].at[0]], gxs[1], gsem)

            @pl.when(t + 1 < nt)
            def _():
                pltpu.async_copy(src_hbm.at[pl.ds(r0 + 2, 1)], srcs[0], isem)
                pltpu.async_copy(dst_hbm.at[pl.ds(r0 + 2, 1)], dsts[0], isem)
            g1.wait()
            s1 = pltpu.async_copy(gxs[1], acc_sh.at[dsts[1].at[0]], ssem, add=True)
            s0.wait()
            s1.wait()
            return carry
        lax.fori_loop(0, nt, block, 0)

    @pl.when(c == 1)
    def _():
        pltpu.async_copy(dst_hbm.at[pl.ds(base, 1)], dsts[0], isem)

        def block(t, carry):
            r0 = base + 2 * t
            pltpu.make_async_copy(dst_hbm.at[pl.ds(r0, 1)], dsts[0], isem).wait()
            i1d = pltpu.async_copy(dst_hbm.at[pl.ds(r0 + 1, 1)], dsts[1], isem)
            g0 = pltpu.async_copy(m_hbm.at[pl.ds(r0 * 128, 128)], gxs[0], gsem)
            g0.wait()
            s0 = pltpu.async_copy(gxs[0], acc_sh.at[dsts[0].at[0]], ssem, add=True)
            i1d.wait()
            g1 = pltpu.async_copy(m_hbm.at[pl.ds((r0 + 1) * 128, 128)], gxs[1], gsem)

            @pl.when(t + 1 < nt)
            def _():
                pltpu.async_copy(dst_hbm.at[pl.ds(r0 + 2, 1)], dsts[0], isem)
            g1.wait()
            s1 = pltpu.async_copy(gxs[1], acc_sh.at[dsts[1].at[0]], ssem, add=True)
            s0.wait()
            s1.wait()
            return carry
        lax.fori_loop(0, nt, block, 0)

    plsc.subcore_barrier()

    def ocopy(i, carry):
        pltpu.sync_copy(acc_sh.at[pl.ds(s * RPS + i * 128, 128)],
                        parts_hbm.at[c, pl.ds(s * RPS + i * 128, 128)])
        return carry
    lax.fori_loop(0, nfull, ocopy, 0)
    pltpu.sync_copy(acc_sh.at[pl.ds(s * RPS + nfull * 128, RPS - nfull * 128)],
                    parts_hbm.at[c, pl.ds(s * RPS + nfull * 128, RPS - nfull * 128)])


def _bdot(a, b):
    return jnp.dot(a.astype(jnp.bfloat16), b.astype(jnp.bfloat16),
                   preferred_element_type=jnp.float32)


def _tc_tab(xp_ref, Winp_ref, bin_ref, t_ref):
    t_ref[...] = _bdot(xp_ref[...], Winp_ref[...]) + bin_ref[...]


def _tc_emb(eavp_ref, Wep_ref, bep_ref, m_ref):
    m_ref[...] = _bdot(eavp_ref[...], Wep_ref[...]) + bep_ref[...]


def _tc_z(parts_ref, t_ref, Wep_ref, bep_ref, W1_ref, b1_ref, z_ref, stats_ref):
    i = pl.program_id(0)
    col = lax.broadcasted_iota(jnp.int32, (1, 16), 1)
    srow = jnp.where(col == 1, 1.0, 0.0).astype(jnp.float32)
    embself = _bdot(srow, Wep_ref[...]) + bep_ref[...]
    aggr = jnp.concatenate(
        [parts_ref[0] + t_ref[...], parts_ref[1] + embself], axis=1)
    z = _bdot(aggr, W1_ref[...]) + b1_ref[...]
    z_ref[...] = z

    @pl.when(i == 0)
    def _():
        stats_ref[...] = jnp.zeros_like(stats_ref)
    stats_ref[0:1, :] += jnp.sum(z, axis=0, keepdims=True)
    stats_ref[1:2, :] += jnp.sum(z * z, axis=0, keepdims=True)


def _tc_norm(relu_out, z_ref, stats_ref, g_ref, bt_ref, W2_ref, b2_ref, o_ref):
    mu = stats_ref[0:1, :] / N
    var = stats_ref[1:2, :] / N - mu * mu
    rstd = lax.rsqrt(var + 1e-5)
    zn = (z_ref[...] - mu) * rstd * g_ref[...] + bt_ref[...]
    zn = jnp.maximum(zn, 0.0)
    o = _bdot(zn, W2_ref[...]) + b2_ref[...]
    if relu_out:
        o = jnp.maximum(o, 0.0)
    o_ref[...] = o


def _full(shape):
    return pl.BlockSpec(shape, lambda i: (0,) * len(shape))


@functools.cache
def _sc_kernels():
    mesh = plsc.VectorSubcoreMesh(core_axis_name="c", subcore_axis_name="s")
    agg = pl.kernel(
        _sc_agg,
        out_type=jax.ShapeDtypeStruct((NC, NP, EMB), jnp.float32),
        mesh=mesh,
        compiler_params=pltpu.CompilerParams(use_tc_tiling_on_sc=False),
        scratch_types=[
            pltpu.VMEM_SHARED((NP, EMB), jnp.float32),
            pltpu.VMEM((2, 128), jnp.int32),
            pltpu.VMEM((2, 128), jnp.int32),
            pltpu.VMEM((256, EMB), jnp.float32),
            pltpu.SemaphoreType.DMA,
            pltpu.SemaphoreType.DMA,
            pltpu.SemaphoreType.DMA,
        ],
    )
    return agg


_GRID = N // RB


def _node_table(xp, Winp, bin2):
    return pl.pallas_call(
        _tc_tab,
        grid=(_GRID,),
        in_specs=[pl.BlockSpec((RB, 16), lambda i: (i, 0)),
                  _full((16, EMB)), _full((1, EMB))],
        out_specs=pl.BlockSpec((RB, EMB), lambda i: (i, 0)),
        out_shape=jax.ShapeDtypeStruct((N, EMB), jnp.float32),
    )(xp, Winp, bin2)


def _edge_table(eavp, Wep, bep):
    return pl.pallas_call(
        _tc_emb,
        grid=(EP // EB,),
        in_specs=[pl.BlockSpec((EB, 16), lambda i: (i, 0)),
                  _full((16, EMB)), _full((1, EMB))],
        out_specs=pl.BlockSpec((EB, EMB), lambda i: (i, 0)),
        out_shape=jax.ShapeDtypeStruct((EP, EMB), jnp.float32),
    )(eavp, Wep, bep)


def _dense(parts, t, Wep, bep, W1, b1, g, bt, W2, b2, relu_out):
    z, stats = pl.pallas_call(
        _tc_z,
        grid=(_GRID,),
        in_specs=[
            pl.BlockSpec((NC, RB, EMB), lambda i: (0, i, 0)),
            pl.BlockSpec((RB, EMB), lambda i: (i, 0)),
            _full((16, EMB)), _full((1, EMB)),
            _full((2 * EMB, 2 * EMB)), _full((1, 2 * EMB)),
        ],
        out_specs=[
            pl.BlockSpec((RB, 2 * EMB), lambda i: (i, 0)),
            _full((2, 2 * EMB)),
        ],
        out_shape=[
            jax.ShapeDtypeStruct((N, 2 * EMB), jnp.float32),
            jax.ShapeDtypeStruct((2, 2 * EMB), jnp.float32),
        ],
    )(parts, t, Wep, bep, W1, b1)
    return pl.pallas_call(
        functools.partial(_tc_norm, relu_out),
        grid=(_GRID,),
        in_specs=[
            pl.BlockSpec((RB, 2 * EMB), lambda i: (i, 0)),
            _full((2, 2 * EMB)),
            _full((1, 2 * EMB)), _full((1, 2 * EMB)),
            _full((2 * EMB, EMB)), _full((1, EMB)),
        ],
        out_specs=pl.BlockSpec((RB, EMB), lambda i: (i, 0)),
        out_shape=jax.ShapeDtypeStruct((N, EMB), jnp.float32),
    )(z, stats, g, bt, W2, b2)


def _pad16(w):
    return jnp.concatenate([w, jnp.zeros((16 - NEF, EMB), jnp.float32)], axis=0)


def kernel(x, edge_index, edge_attr, Win, bin_, We0, be0, W10, b10, g0, bt0,
           W20, b20, We1, be1, W11, b11, g1, bt1, W21, b21):
    f32 = jnp.float32
    pad = EP - E
    lane = jnp.arange(pad, dtype=jnp.int32) % 16
    src = jnp.concatenate([edge_index[0], lane]).reshape(EROWS, 128)
    dst = jnp.concatenate([edge_index[1], N + lane]).reshape(EROWS, 128)
    xp = jnp.concatenate([x, jnp.zeros((N, 16 - NEF), f32)], axis=1)
    eavp = jnp.concatenate([
        jnp.concatenate([edge_attr, jnp.zeros((E, 16 - NEF), f32)], axis=1),
        jnp.zeros((pad, 16), f32)], axis=0)

    agg = _sc_kernels()
    t0 = _node_table(xp, _pad16(Win), bin_.reshape(1, EMB))
    m0 = _edge_table(eavp, _pad16(We0), be0.reshape(1, EMB))
    parts0 = agg(t0, m0, src, dst)[:, :N, :]
    h1 = _dense(parts0, t0, _pad16(We0), be0.reshape(1, EMB), W10,
                b10.reshape(1, 2 * EMB), g0.reshape(1, 2 * EMB),
                bt0.reshape(1, 2 * EMB), W20, b20.reshape(1, EMB), True)
    m1 = _edge_table(eavp, _pad16(We1), be1.reshape(1, EMB))
    parts1 = agg(h1, m1, src, dst)[:, :N, :]
    out = _dense(parts1, h1, _pad16(We1), be1.reshape(1, EMB), W11,
                 b11.reshape(1, 2 * EMB), g1.reshape(1, 2 * EMB),
                 bt1.reshape(1, 2 * EMB), W21, b21.reshape(1, EMB), False)
    return out


# speedup vs baseline: 1.4977x; 1.1028x over previous
import functools

import jax
import jax.numpy as jnp
from jax import lax
from jax.experimental import pallas as pl
from jax.experimental.pallas import tpu as pltpu
from jax.experimental.pallas import tpu_sc as plsc

N = 10000
E = 320000
EMB = 128
NEF = 3

NC = 2
NS = 16
NW = NC * NS

SINK = 112
NP = N + SINK
RPS = NP // NS

EP = 327680
EROWS = EP // 128
RPT = EROWS // NS

RB = 1000
EB = 2048


def _sc_agg(t_hbm, m_hbm, src_hbm, dst_hbm, parts_hbm,
            acc_sh, srcb, dstb, gxb, isem, gsem, ssem0, ssem1):
    c = lax.axis_index("c")
    s = lax.axis_index("s")

    def zrow(i, carry):
        for jj in range(8):
            gxb[i, pl.ds(jj * 16, 16)] = jnp.zeros((16,), jnp.float32)
        return carry
    lax.fori_loop(0, 128, zrow, 0)
    nfull = RPS // 128
    def zcopy(i, carry):
        pltpu.sync_copy(gxb.at[pl.ds(0, 128)],
                        acc_sh.at[pl.ds(s * RPS + i * 128, 128)])
        return carry
    lax.fori_loop(0, nfull, zcopy, 0)
    pltpu.sync_copy(gxb.at[pl.ds(0, RPS - nfull * 128)],
                    acc_sh.at[pl.ds(s * RPS + nfull * 128, RPS - nfull * 128)])
    plsc.subcore_barrier()

    base = s * RPT
    nt = RPT // 2
    srcs = (srcb.at[pl.ds(0, 1)], srcb.at[pl.ds(1, 1)])
    dsts = (dstb.at[pl.ds(0, 1)], dstb.at[pl.ds(1, 1)])
    gxs = (gxb.at[pl.ds(0, 128)], gxb.at[pl.ds(128, 128)])

    ssems = (ssem0, ssem1)

    def drain_scatter(k):
        pltpu.make_async_copy(gxs[k], acc_sh.at[pl.ds(0, 128)], ssems[k]).wait()

    @pl.when(c == 0)
    def _():
        pltpu.async_copy(src_hbm.at[pl.ds(base, 1)], srcs[0], isem)
        pltpu.async_copy(dst_hbm.at[pl.ds(base, 1)], dsts[0], isem)

        def block(t, carry):
            r0 = base + 2 * t
            pltpu.make_async_copy(src_hbm.at[pl.ds(r0, 1)], srcs[0], isem).wait()
            pltpu.make_async_copy(dst_hbm.at[pl.ds(r0, 1)], dsts[0], isem).wait()
            i1s = pltpu.async_copy(src_hbm.at[pl.ds(r0 + 1, 1)], srcs[1], isem)
            i1d = pltpu.async_copy(dst_hbm.at[pl.ds(r0 + 1, 1)], dsts[1], isem)

            @pl.when(t > 0)
            def _():
                drain_scatter(0)
            g0 = pltpu.async_copy(t_hbm.at[srcs[0].at[0]], gxs[0], gsem)
            g0.wait()
            pltpu.async_copy(gxs[0], acc_sh.at[dsts[0].at[0]], ssem0, add=True)
            i1s.wait()
            i1d.wait()

            @pl.when(t > 0)
            def _():
                drain_scatter(1)
            g1 = pltpu.async_copy(t_hbm.at[srcs[1].at[0]], gxs[1], gsem)

            @pl.when(t + 1 < nt)
            def _():
                pltpu.async_copy(src_hbm.at[pl.ds(r0 + 2, 1)], srcs[0], isem)
                pltpu.async_copy(dst_hbm.at[pl.ds(r0 + 2, 1)], dsts[0], isem)
            g1.wait()
            pltpu.async_copy(gxs[1], acc_sh.at[dsts[1].at[0]], ssem1, add=True)
            return carry
        lax.fori_loop(0, nt, block, 0)

    @pl.when(c == 1)
    def _():
        pltpu.async_copy(dst_hbm.at[pl.ds(base, 1)], dsts[0], isem)

        def block(t, carry):
            r0 = base + 2 * t
            pltpu.make_async_copy(dst_hbm.at[pl.ds(r0, 1)], dsts[0], isem).wait()
            i1d = pltpu.async_copy(dst_hbm.at[pl.ds(r0 + 1, 1)], dsts[1], isem)

            @pl.when(t > 0)
            def _():
                drain_scatter(0)
            g0 = pltpu.async_copy(m_hbm.at[pl.ds(r0 * 128, 128)], gxs[0], gsem)
            g0.wait()
            pltpu.async_copy(gxs[0], acc_sh.at[dsts[0].at[0]], ssem0, add=True)
            i1d.wait()

            @pl.when(t > 0)
            def _():
                drain_scatter(1)
            g1 = pltpu.async_copy(m_hbm.at[pl.ds((r0 + 1) * 128, 128)], gxs[1], gsem)

            @pl.when(t + 1 < nt)
            def _():
                pltpu.async_copy(dst_hbm.at[pl.ds(r0 + 2, 1)], dsts[0], isem)
            g1.wait()
            pltpu.async_copy(gxs[1], acc_sh.at[dsts[1].at[0]], ssem1, add=True)
            return carry
        lax.fori_loop(0, nt, block, 0)

    drain_scatter(0)
    drain_scatter(1)
    plsc.subcore_barrier()

    def ocopy(i, carry):
        pltpu.sync_copy(acc_sh.at[pl.ds(s * RPS + i * 128, 128)],
                        parts_hbm.at[c, pl.ds(s * RPS + i * 128, 128)])
        return carry
    lax.fori_loop(0, nfull, ocopy, 0)
    pltpu.sync_copy(acc_sh.at[pl.ds(s * RPS + nfull * 128, RPS - nfull * 128)],
                    parts_hbm.at[c, pl.ds(s * RPS + nfull * 128, RPS - nfull * 128)])


def _bdot(a, b):
    return jnp.dot(a.astype(jnp.bfloat16), b.astype(jnp.bfloat16),
                   preferred_element_type=jnp.float32)


def _tc_tab(x_ref, Win_ref, bin_ref, t_ref):
    t_ref[...] = _bdot(x_ref[...], Win_ref[...]) + bin_ref[...]


def _tc_emb(eap_ref, We_ref, be_ref, m_ref):
    m_ref[...] = _bdot(eap_ref[...], We_ref[...]) + be_ref[...]


def _tc_layer(relu_out, parts_ref, t_ref, We_ref, be_ref, W1_ref, b1_ref,
              g_ref, bt_ref, W2_ref, b2_ref, o_ref, z_vm, stats_vm):
    p = pl.program_id(0)
    i = pl.program_id(1)

    @pl.when(p == 0)
    def _():
        col = lax.broadcasted_iota(jnp.int32, (1, NEF), 1)
        srow = jnp.where(col == 1, 1.0, 0.0).astype(jnp.float32)
        embself = _bdot(srow, We_ref[...]) + be_ref[...]
        aggr = jnp.concatenate(
            [parts_ref[0] + t_ref[...], parts_ref[1] + embself], axis=1)
        z = _bdot(aggr, W1_ref[...]) + b1_ref[...]
        z_vm[pl.ds(i * RB, RB), :] = z

        @pl.when(i == 0)
        def _():
            stats_vm[...] = jnp.zeros_like(stats_vm)
        stats_vm[0:1, :] += jnp.sum(z, axis=0, keepdims=True)
        stats_vm[1:2, :] += jnp.sum(z * z, axis=0, keepdims=True)

    @pl.when(p == 1)
    def _():
        mu = stats_vm[0:1, :] / N
        var = stats_vm[1:2, :] / N - mu * mu
        rstd = lax.rsqrt(var + 1e-5)
        zn = (z_vm[pl.ds(i * RB, RB), :] - mu) * rstd * g_ref[...] + bt_ref[...]
        zn = jnp.maximum(zn, 0.0)
        o = _bdot(zn, W2_ref[...]) + b2_ref[...]
        if relu_out:
            o = jnp.maximum(o, 0.0)
        o_ref[...] = o


def _full(shape):
    return pl.BlockSpec(shape, lambda *_: (0,) * len(shape))


@functools.cache
def _sc_kernels():
    mesh = plsc.VectorSubcoreMesh(core_axis_name="c", subcore_axis_name="s")
    agg = pl.kernel(
        _sc_agg,
        out_type=jax.ShapeDtypeStruct((NC, NP, EMB), jnp.float32),
        mesh=mesh,
        compiler_params=pltpu.CompilerParams(use_tc_tiling_on_sc=False),
        scratch_types=[
            pltpu.VMEM_SHARED((NP, EMB), jnp.float32),
            pltpu.VMEM((2, 128), jnp.int32),
            pltpu.VMEM((2, 128), jnp.int32),
            pltpu.VMEM((256, EMB), jnp.float32),
            pltpu.SemaphoreType.DMA,
            pltpu.SemaphoreType.DMA,
            pltpu.SemaphoreType.DMA,
            pltpu.SemaphoreType.DMA,
        ],
    )
    return agg


_GRID = N // RB


def _node_table(x, Win, bin2):
    return pl.pallas_call(
        _tc_tab,
        grid=(_GRID,),
        in_specs=[pl.BlockSpec((RB, NEF), lambda i: (i, 0)),
                  _full((NEF, EMB)), _full((1, EMB))],
        out_specs=pl.BlockSpec((RB, EMB), lambda i: (i, 0)),
        out_shape=jax.ShapeDtypeStruct((N, EMB), jnp.float32),
    )(x, Win, bin2)


def _edge_table(eap, We, bep):
    return pl.pallas_call(
        _tc_emb,
        grid=(EP // EB,),
        in_specs=[pl.BlockSpec((EB, NEF), lambda i: (i, 0)),
                  _full((NEF, EMB)), _full((1, EMB))],
        out_specs=pl.BlockSpec((EB, EMB), lambda i: (i, 0)),
        out_shape=jax.ShapeDtypeStruct((EP, EMB), jnp.float32),
    )(eap, We, bep)


def _dense(parts, t, We, bep, W1, b1, g, bt, W2, b2, relu_out):
    return pl.pallas_call(
        functools.partial(_tc_layer, relu_out),
        grid=(2, _GRID),
        in_specs=[
            pl.BlockSpec((NC, RB, EMB), lambda p, i: (0, i, 0)),
            pl.BlockSpec((RB, EMB), lambda p, i: (i, 0)),
            _full((NEF, EMB)), _full((1, EMB)),
            _full((2 * EMB, 2 * EMB)), _full((1, 2 * EMB)),
            _full((1, 2 * EMB)), _full((1, 2 * EMB)),
            _full((2 * EMB, EMB)), _full((1, EMB)),
        ],
        out_specs=pl.BlockSpec((RB, EMB), lambda p, i: (p * i, 0)),
        out_shape=jax.ShapeDtypeStruct((N, EMB), jnp.float32),
        scratch_shapes=[
            pltpu.VMEM((N, 2 * EMB), jnp.float32),
            pltpu.VMEM((2, 2 * EMB), jnp.float32),
        ],
    )(parts, t, We, bep, W1, b1, g, bt, W2, b2)


def kernel(x, edge_index, edge_attr, Win, bin_, We0, be0, W10, b10, g0, bt0,
           W20, b20, We1, be1, W11, b11, g1, bt1, W21, b21):
    f32 = jnp.float32
    pad = EP - E
    lane = jnp.arange(pad, dtype=jnp.int32) % 16
    src = jnp.concatenate([edge_index[0], lane]).reshape(EROWS, 128)
    dst = jnp.concatenate([edge_index[1], N + lane]).reshape(EROWS, 128)
    eap = jnp.concatenate([edge_attr, jnp.zeros((pad, NEF), f32)], axis=0)

    agg = _sc_kernels()
    t0 = _node_table(x, Win, bin_.reshape(1, EMB))
    m0 = _edge_table(eap, We0, be0.reshape(1, EMB))
    parts0 = agg(t0, m0, src, dst)
    h1 = _dense(parts0, t0, We0, be0.reshape(1, EMB), W10,
                b10.reshape(1, 2 * EMB), g0.reshape(1, 2 * EMB),
                bt0.reshape(1, 2 * EMB), W20, b20.reshape(1, EMB), True)
    m1 = _edge_table(eap, We1, be1.reshape(1, EMB))
    parts1 = agg(h1, m1, src, dst)
    out = _dense(parts1, h1, We1, be1.reshape(1, EMB), W11,
                 b11.reshape(1, 2 * EMB), g1.reshape(1, 2 * EMB),
                 bt1.reshape(1, 2 * EMB), W21, b21.reshape(1, EMB), False)
    return out
